# Initial kernel scaffold; baseline (speedup 1.0000x reference)
import jax
import jax.numpy as jnp
from jax.experimental import pallas as pl


def kernel(x, pos, key_padding_mask, valid_sizes, valid_scales, W_loc, b_loc, W_wt, b_wt, W_val, b_val, W_out, b_out, scale):
    raise NotImplementedError("write your pallas kernel here")


# speedup vs baseline: 1.0794x; 1.0794x over previous
import functools

import jax
import jax.numpy as jnp
import numpy as np
from jax.experimental import pallas as pl
from jax.experimental.pallas import tpu as pltpu

_D = 512
_M = 8
_P = 4
_F = 4
_SHAPES = ((48, 48), (24, 24), (12, 12), (6, 6))
_S = sum(h * w for h, w in _SHAPES)
_SP = 3072


def _mm_body(w_ref, x_ref, b_ref, o_ref):
    o_ref[0] = (
        jnp.dot(w_ref[...], x_ref[0], preferred_element_type=jnp.float32)
        + b_ref[...]
    )


def _conv1x1(W, b, x):
    Cout = W.shape[0]
    N = x.shape[0]
    return pl.pallas_call(
        _mm_body,
        grid=(N, _SP // 512),
        in_specs=[
            pl.BlockSpec((Cout, _D), lambda n, s: (0, 0)),
            pl.BlockSpec((1, _D, 512), lambda n, s: (n, 0, s)),
            pl.BlockSpec((Cout, 1), lambda n, s: (0, 0)),
        ],
        out_specs=pl.BlockSpec((1, Cout, 512), lambda n, s: (n, 0, s)),
        out_shape=jax.ShapeDtypeStruct((N, Cout, _SP), jnp.float32),
    )(W, x, b.reshape(Cout, 1))


def _grid_sample(value, grid):
    B, C, H, W = value.shape
    x = (grid[..., 0] + 1.0) * 0.5 * W - 0.5
    y = (grid[..., 1] + 1.0) * 0.5 * H - 0.5
    x0 = jnp.floor(x)
    y0 = jnp.floor(y)
    x1 = x0 + 1.0
    y1 = y0 + 1.0
    flat = value.reshape(B, C, H * W)

    def corner(xi, yi, w):
        valid = ((xi >= 0) & (xi <= W - 1) & (yi >= 0) & (yi <= H - 1)).astype(value.dtype)
        xi_c = jnp.clip(xi, 0, W - 1).astype(jnp.int32)
        yi_c = jnp.clip(yi, 0, H - 1).astype(jnp.int32)
        idx = yi_c * W + xi_c
        g = jax.vmap(lambda v, i: v[:, i])(flat, idx)
        return g * (w * valid)[:, None, :, :]

    return (corner(x0, y0, (x1 - x) * (y1 - y))
            + corner(x1, y0, (x - x0) * (y1 - y))
            + corner(x0, y1, (x1 - x) * (y - y0))
            + corner(x1, y1, (x - x0) * (y - y0)))


def kernel(x, pos, key_padding_mask, valid_sizes, valid_scales, W_loc, b_loc,
           W_wt, b_wt, W_val, b_val, W_out, b_out, scale):
    N, C, S = x.shape
    maskf = key_padding_mask.reshape(N, 1, S).astype(x.dtype)
    keep = 1.0 - maskf
    x_pos = (x + pos) * keep
    pad = ((0, 0), (0, 0), (0, _SP - S))
    xP = jnp.pad(x, pad)
    xpP = jnp.pad(x_pos, pad)

    value = _conv1x1(W_val, b_val, xP)[..., :S] * keep
    loc = _conv1x1(W_loc, b_loc, xpP)[..., :S]
    wtl = _conv1x1(W_wt, b_wt, xpP)[..., :S]

    vs = jnp.repeat(valid_sizes.reshape(N, _F, 1, 1, 2), _M, axis=0)
    vsc = 2.0 * jnp.repeat(valid_scales.reshape(N, _F, 1, 1, 2), _M, axis=0)

    values = []
    cur = 0
    for (H_, W_) in _SHAPES:
        hw = H_ * W_
        values.append(value[..., cur:cur + hw].reshape(N * _M, C // _M, H_, W_))
        cur += hw

    outs = []
    cur = 0
    for lvl, (H_, W_) in enumerate(_SHAPES):
        hw = H_ * W_
        offsets = (loc[..., cur:cur + hw].reshape(N * _M, _F, _P, 2, hw)
                   .transpose(0, 1, 2, 4, 3))
        weights = jax.nn.softmax(
            wtl[..., cur:cur + hw].reshape(N * _M, 1, _F * _P, hw), axis=2)
        gy, gx = jnp.meshgrid(jnp.linspace(0.5, H_ - 0.5, H_),
                              jnp.linspace(0.5, W_ - 0.5, W_), indexing='ij')
        pre = jnp.stack([gx, gy], axis=-1).reshape(hw, 2)
        sc = vsc / vs[:, lvl].reshape(N * _M, 1, 1, 1, 2)
        grids = offsets * sc + (pre.reshape(1, 1, 1, hw, 2) * sc - 1.0)
        grids = jnp.transpose(grids, (1, 0, 2, 3, 4))
        samples = [_grid_sample(values[f], grids[f]) for f in range(_F)]
        out = jnp.sum(jnp.concatenate(samples, axis=2) * weights, axis=2).reshape(N, C, hw)
        outs.append(out)
        cur += hw

    gathered = jnp.concatenate(outs, axis=-1)
    gatheredP = jnp.pad(gathered, pad)
    final = _conv1x1(W_out, b_out, gatheredP)[..., :S]
    return final * scale.reshape(1, -1, 1)


# speedup vs baseline: 31.3945x; 29.0844x over previous
import functools

import jax
import jax.numpy as jnp
import numpy as np
from jax import lax
from jax.experimental import pallas as pl
from jax.experimental.pallas import tpu as pltpu
from jax.experimental.pallas import tpu_sc as plsc

_D = 512
_M = 8
_P = 4
_F = 4
_SHAPES = ((48, 48), (24, 24), (12, 12), (6, 6))
_S = sum(h * w for h, w in _SHAPES)
_SP = 3072
_NW = 32
_QC = 128
_NCHUNK = _SP // _QC
_J = _F * _P * 4

_LEVEL_BASE = []
_cur = 0
for _h, _w in _SHAPES:
    _LEVEL_BASE.append(_cur)
    _cur += _h * _w


def _mm_body(w_ref, x_ref, b_ref, o_ref):
    o_ref[0] = (
        jnp.dot(w_ref[...], x_ref[0], preferred_element_type=jnp.float32)
        + b_ref[...]
    )


def _conv1x1(W, b, x):
    Cout = W.shape[0]
    N = x.shape[0]
    return pl.pallas_call(
        _mm_body,
        grid=(N, _SP // 512),
        in_specs=[
            pl.BlockSpec((Cout, _D), lambda n, s: (0, 0)),
            pl.BlockSpec((1, _D, 512), lambda n, s: (n, 0, s)),
            pl.BlockSpec((Cout, 1), lambda n, s: (0, 0)),
        ],
        out_specs=pl.BlockSpec((1, Cout, 512), lambda n, s: (n, 0, s)),
        out_shape=jax.ShapeDtypeStruct((N, Cout, _SP), jnp.float32),
    )(W, x, b.reshape(Cout, 1))


def _query_consts():
    cx, cy, lvl = [], [], []
    for l, (H, W) in enumerate(_SHAPES):
        gy, gx = np.meshgrid(np.linspace(0.5, H - 0.5, H),
                             np.linspace(0.5, W - 0.5, W), indexing='ij')
        cx.append(gx.ravel())
        cy.append(gy.ravel())
        lvl.append(np.full(H * W, l, np.int32))
    return (np.concatenate(cx).astype(np.float32),
            np.concatenate(cy).astype(np.float32),
            np.concatenate(lvl))


_CX, _CY, _LVL = _query_consts()


def _index_weights(loc, attn, valid_sizes, valid_scales):
    N = loc.shape[0]
    off = loc.reshape(N, _M, _F, _P, 2, _S)
    cx = jnp.asarray(_CX)
    cy = jnp.asarray(_CY)
    szf = jnp.array([[w, h] for (h, w) in _SHAPES], jnp.float32)
    denom = valid_sizes[:, _LVL, :]
    A = (valid_scales[:, :, None, :] * szf[None, :, None, :]) / denom[:, None, :, :]
    Ax = A[..., 0][:, None, :, None, :]
    Ay = A[..., 1][:, None, :, None, :]
    u = (off[:, :, :, :, 0, :] + cx) * Ax - 0.5
    v = (off[:, :, :, :, 1, :] + cy) * Ay - 0.5
    x0 = jnp.floor(u)
    y0 = jnp.floor(v)
    fx = u - x0
    fy = v - y0
    idx_parts, w_parts = [], []
    for f in range(_F):
        H_, W_ = _SHAPES[f]
        base = _LEVEL_BASE[f]
        x0f, y0f = x0[:, :, f], y0[:, :, f]
        fxf, fyf = fx[:, :, f], fy[:, :, f]
        aw = attn[:, :, f * _P:(f + 1) * _P, :]
        i4, w4 = [], []
        for dy in (0, 1):
            yi = y0f + dy
            wy = fyf if dy else 1.0 - fyf
            vy = (yi >= 0) & (yi <= H_ - 1)
            yc = jnp.clip(yi, 0, H_ - 1).astype(jnp.int32)
            for dx in (0, 1):
                xi = x0f + dx
                wx = fxf if dx else 1.0 - fxf
                vx = (xi >= 0) & (xi <= W_ - 1)
                xc = jnp.clip(xi, 0, W_ - 1).astype(jnp.int32)
                i4.append(base + yc * W_ + xc)
                w4.append(aw * wx * wy * (vx & vy).astype(jnp.float32))
        idx_parts.append(jnp.stack(i4, axis=3))
        w_parts.append(jnp.stack(w4, axis=3))
    idx = jnp.stack(idx_parts, axis=2)
    w = jnp.stack(w_parts, axis=2)
    return (idx.reshape(N * _M, _J, _S), w.reshape(N * _M, _J, _S))


@functools.lru_cache(maxsize=1)
def _sc_gather_build():
    mesh = plsc.VectorSubcoreMesh(core_axis_name="c", subcore_axis_name="s")

    @functools.partial(
        pl.kernel,
        mesh=mesh,
        compiler_params=pltpu.CompilerParams(needs_layout_passes=False),
        out_type=jax.ShapeDtypeStruct((_M * 2, 64, _SP), jnp.float32),
        scratch_types=[
            pltpu.VMEM((_S * 16,), jnp.float32),
            pltpu.VMEM((_J, _QC), jnp.int32),
            pltpu.VMEM((_J, _QC), jnp.float32),
            pltpu.VMEM((16, _QC), jnp.float32),
        ],
    )
    def sc_gather(tab_hbm, idx_hbm, w_hbm, out_hbm, tab_v, idx_v, w_v, out_v):
        wid = lax.axis_index("s") * 2 + lax.axis_index("c")
        cols = [jnp.full((16,), c, jnp.int32) for c in range(16)]

        def run_task(t):
            nm = t // 4
            cg = t % 4
            pltpu.sync_copy(tab_hbm.at[t], tab_v)

            def chunk_body(ch, _):
                q0 = ch * _QC
                pltpu.sync_copy(idx_hbm.at[nm, :, pl.ds(q0, _QC)], idx_v)
                pltpu.sync_copy(w_hbm.at[nm, :, pl.ds(q0, _QC)], w_v)

                def qg_body(qg, _):
                    qo = qg * 16

                    def j_body(j, accs):
                        ridx = idx_v[j, pl.ds(qo, 16)]
                        wv = w_v[j, pl.ds(qo, 16)]
                        base = ridx * 16
                        new = []
                        for c in range(16):
                            g = plsc.load_gather(tab_v, [base + cols[c]])
                            new.append(accs[c] + g * wv)
                        return tuple(new)

                    accs = lax.fori_loop(
                        0, _J, j_body,
                        tuple(jnp.zeros((16,), jnp.float32) for _ in range(16)))
                    for c in range(16):
                        out_v[c, pl.ds(qo, 16)] = accs[c]
                    return _

                lax.fori_loop(0, _QC // 16, qg_body, 0)
                pltpu.sync_copy(
                    out_v, out_hbm.at[nm, pl.ds(cg * 16, 16), pl.ds(q0, _QC)])
                return _

            lax.fori_loop(0, _NCHUNK, chunk_body, 0)

        def half_body(h, _):
            run_task(wid + h * _NW)
            return _

        lax.fori_loop(0, 2, half_body, 0)

    return sc_gather


def kernel(x, pos, key_padding_mask, valid_sizes, valid_scales, W_loc, b_loc,
           W_wt, b_wt, W_val, b_val, W_out, b_out, scale):
    N, C, S = x.shape
    keep = 1.0 - key_padding_mask.reshape(N, 1, S).astype(x.dtype)
    x_pos = (x + pos) * keep
    pad = ((0, 0), (0, 0), (0, _SP - S))
    xP = jnp.pad(x, pad)
    xpP = jnp.pad(x_pos, pad)

    value = _conv1x1(W_val, b_val, xP)[..., :S] * keep
    loc = _conv1x1(W_loc, b_loc, xpP)[..., :S]
    wtl = _conv1x1(W_wt, b_wt, xpP)[..., :S]

    attn = jax.nn.softmax(wtl.reshape(N, _M, _F * _P, S), axis=2)
    idx, w = _index_weights(loc, attn, valid_sizes, valid_scales)
    idxP = jnp.pad(idx, ((0, 0), (0, 0), (0, _SP - S)))
    wP = jnp.pad(w, ((0, 0), (0, 0), (0, _SP - S)))

    tab = (value.reshape(N, _M, 4, 16, S)
           .transpose(0, 1, 2, 4, 3)
           .reshape(N * _M * 4, S * 16))

    out = _sc_gather_build()(tab, idxP, wP)
    gathered = out[..., :S].reshape(N, C, S)

    gatheredP = jnp.pad(gathered, pad)
    final = _conv1x1(W_out, b_out, gatheredP)[..., :S]
    return final * scale.reshape(1, -1, 1)
